# Initial kernel scaffold; baseline (speedup 1.0000x reference)
import jax
import jax.numpy as jnp
from jax.experimental import pallas as pl


def kernel(x, fwd_edges_index, bwd_edges_index, edge_attr, g0f_W, g0f_as, g0f_ad, g0f_eW, g0f_ae, g0f_b, g0b_W, g0b_as, g0b_ad, g0b_eW, g0b_ae, g0b_b, m0_W, m0_b, bn0_g, bn0_b, r0_W, r0_b, g1f_W, g1f_as, g1f_ad, g1f_eW, g1f_ae, g1f_b, g1b_W, g1b_as, g1b_ad, g1b_eW, g1b_ae, g1b_b, m1_W, m1_b, bn1_g, bn1_b, r1_W, r1_b):
    raise NotImplementedError("write your pallas kernel here")


# speedup vs baseline: 23.2501x; 23.2501x over previous
import functools

import jax
import jax.numpy as jnp
from jax import lax
from jax.experimental import pallas as pl
from jax.experimental.pallas import tpu as pltpu
from jax.experimental.pallas import tpu_sc as plsc

N = 10000
E = 160000
HEADS = 2
NEG = 0.2
EPS = 1e-5
C = 128
HC = 256

NS = 16
ET = E // NS
K = 64
NCH = 160
ETP = NCH * K
DN = 640
F32 = jnp.float32


def _xl_body(x_ref, wf_ref, wb_ref, f0_ref, f1_ref, b0_ref, b1_ref):
    xb = x_ref[...]
    rf = jnp.dot(xb, wf_ref[...], preferred_element_type=F32, precision=lax.Precision.HIGHEST)
    rb = jnp.dot(xb, wb_ref[...], preferred_element_type=F32, precision=lax.Precision.HIGHEST)
    f0_ref[...] = rf[:, :C]
    f1_ref[...] = rf[:, C:]
    b0_ref[...] = rb[:, :C]
    b1_ref[...] = rb[:, C:]


def _xl_proj(x, wf, wb):
    din = x.shape[1]
    bn = 2000
    grid = (N // bn,)
    spec_x = pl.BlockSpec((bn, din), lambda i: (i, 0))
    spec_w = pl.BlockSpec((din, HC), lambda i: (0, 0))
    spec_o = pl.BlockSpec((bn, C), lambda i: (i, 0))
    out = jax.ShapeDtypeStruct((N, C), F32)
    return pl.pallas_call(
        _xl_body,
        grid=grid,
        in_specs=[spec_x, spec_w, spec_w],
        out_specs=[spec_o] * 4,
        out_shape=[out] * 4,
    )(x, wf, wb)


def _attn_body(x_ref, wf_ref, asf_ref, adf_ref, wb_ref, asb_ref, adb_ref,
               out_ref):
    din = x_ref.shape[1]

    def fold(w_ref, a_ref):
        w3 = w_ref[...].reshape(din, HEADS, C)
        return (w3 * a_ref[...][None]).sum(-1)

    proj = jnp.concatenate(
        [fold(wf_ref, asf_ref), fold(wf_ref, adf_ref),
         fold(wb_ref, asb_ref), fold(wb_ref, adb_ref)], axis=1)
    out_ref[...] = lax.dot_general(
        proj, x_ref[...], (((0,), (1,)), ((), ())),
        preferred_element_type=F32, precision=lax.Precision.HIGHEST)


def _attn_proj(x, wf, asf, adf, wb, asb, adb):
    din = x.shape[1]
    full2 = lambda s: pl.BlockSpec(s, lambda: (0, 0))
    return pl.pallas_call(
        _attn_body,
        in_specs=[full2((N, din)), full2((din, HC)), full2((HEADS, C)),
                  full2((HEADS, C)), full2((din, HC)), full2((HEADS, C)),
                  full2((HEADS, C))],
        out_specs=full2((8, N)),
        out_shape=jax.ShapeDtypeStruct((8, N), F32),
    )(x, wf, asf, adf, wb, asb, adb)


def _edge_body(ea_ref, *rest):
    ws = rest[:8]
    out_ref = rest[8]
    projs = []
    for g in range(4):
        ew3 = ws[2 * g][...].reshape(4, HEADS, C)
        projs.append((ew3 * ws[2 * g + 1][...][None]).sum(-1))
    projall = jnp.concatenate(projs, axis=1)
    out_ref[...] = lax.dot_general(
        projall, ea_ref[...], (((0,), (1,)), ((), ())),
        preferred_element_type=F32, precision=lax.Precision.HIGHEST)


def _edge_attn(ea, *ew_ae):
    be = 16000
    grid = (E // be,)
    specs = [pl.BlockSpec((be, 4), lambda i: (i, 0))]
    for _ in range(4):
        specs.append(pl.BlockSpec((4, HC), lambda i: (0, 0)))
        specs.append(pl.BlockSpec((HEADS, C), lambda i: (0, 0)))
    return pl.pallas_call(
        _edge_body,
        grid=grid,
        in_specs=specs,
        out_specs=pl.BlockSpec((8, be), lambda i: (0, i)),
        out_shape=jax.ShapeDtypeStruct((8, E), F32),
    )(ea, *ew_ae)


def _merge_a_body(f0_ref, f1_ref, b0_ref, b1_ref, gfb_ref, gbb_ref,
                  mw_ref, mb_ref, h_ref, st_ref):
    i = pl.program_id(0)
    mw = mw_ref[...]
    hv = (jnp.dot(f0_ref[...], mw[0:C], preferred_element_type=F32, precision=lax.Precision.HIGHEST)
          + jnp.dot(f1_ref[...], mw[C:2 * C], preferred_element_type=F32, precision=lax.Precision.HIGHEST)
          + jnp.dot(b0_ref[...], mw[2 * C:3 * C], preferred_element_type=F32, precision=lax.Precision.HIGHEST)
          + jnp.dot(b1_ref[...], mw[3 * C:], preferred_element_type=F32, precision=lax.Precision.HIGHEST))
    be = (jnp.dot(gfb_ref[...].reshape(1, HC), mw[:HC],
                  preferred_element_type=F32, precision=lax.Precision.HIGHEST)
          + jnp.dot(gbb_ref[...].reshape(1, HC), mw[HC:],
                    preferred_element_type=F32, precision=lax.Precision.HIGHEST))
    hv = jnp.maximum(hv + be + mb_ref[...][None, :], 0.0)
    h_ref[...] = hv

    @pl.when(i == 0)
    def _():
        st_ref[...] = jnp.zeros((2, HC), F32)

    st_ref[0, :] = st_ref[0, :] + hv.sum(axis=0)
    st_ref[1, :] = st_ref[1, :] + (hv * hv).sum(axis=0)


def _merge_a(f0, f1, b0, b1, gfb, gbb, mw, mb):
    bn = 2000
    grid = (N // bn,)
    spec_h = pl.BlockSpec((bn, C), lambda i: (i, 0))
    full = lambda s: pl.BlockSpec(s, lambda i: tuple(0 for _ in s))
    return pl.pallas_call(
        _merge_a_body,
        grid=grid,
        in_specs=[spec_h, spec_h, spec_h, spec_h, full((HC,)), full((HC,)),
                  full((2 * HC, HC)), full((HC,))],
        out_specs=[pl.BlockSpec((bn, HC), lambda i: (i, 0)),
                   full((2, HC))],
        out_shape=[jax.ShapeDtypeStruct((N, HC), F32),
                   jax.ShapeDtypeStruct((2, HC), F32)],
    )(f0, f1, b0, b1, gfb, gbb, mw, mb)


def _merge_b_body_res(h_ref, st_ref, g_ref, b_ref, x0_ref, rw_ref, rb_ref,
                      out_ref):
    mu = st_ref[0, :] * (1.0 / N)
    var = st_ref[1, :] * (1.0 / N) - mu * mu
    scale = g_ref[...] * lax.rsqrt(var + EPS)
    shift = b_ref[...] - mu * scale
    res = jnp.dot(x0_ref[...], rw_ref[...], preferred_element_type=F32, precision=lax.Precision.HIGHEST)
    out_ref[...] = h_ref[...] * scale[None, :] + shift[None, :] \
        + res + rb_ref[...][None, :]


def _merge_b_body_id(h_ref, st_ref, g_ref, b_ref, x0_ref, out_ref):
    mu = st_ref[0, :] * (1.0 / N)
    var = st_ref[1, :] * (1.0 / N) - mu * mu
    scale = g_ref[...] * lax.rsqrt(var + EPS)
    shift = b_ref[...] - mu * scale
    out_ref[...] = h_ref[...] * scale[None, :] + shift[None, :] + x0_ref[...]


def _merge_b(h, st, g, b, x0, rw=None, rb=None):
    bn = 2000
    din = x0.shape[1]
    grid = (N // bn,)
    spec_h = pl.BlockSpec((bn, HC), lambda i: (i, 0))
    full = lambda s: pl.BlockSpec(s, lambda i: tuple(0 for _ in s))
    args = [h, st, g, b, x0]
    specs = [spec_h, full((2, HC)), full((HC,)), full((HC,)),
             pl.BlockSpec((bn, din), lambda i: (i, 0))]
    if rw is not None:
        body = _merge_b_body_res
        args += [rw, rb]
        specs += [full((din, HC)), full((HC,))]
    else:
        body = _merge_b_body_id
    return pl.pallas_call(
        body,
        grid=grid,
        in_specs=specs,
        out_specs=spec_h,
        out_shape=jax.ShapeDtypeStruct((N, HC), F32),
    )(*args)


def _att_sc_body(d, g, attn, aeT, src, dst, out,
                 src_t, dst_t, ae_t, as_t, ad_t, den_t, io_t, sh_den):
    h = lax.axis_index("c")
    tid = lax.axis_index("s")
    e0 = tid * ET

    pltpu.sync_copy(src.at[pl.ds(e0, ET)], src_t)
    pltpu.sync_copy(dst.at[pl.ds(e0, ET)], dst_t)
    pltpu.sync_copy(aeT.at[pl.ds((2 * g + h) * E + e0, ET)], ae_t)
    pltpu.sync_copy(attn.at[pl.ds((4 * d + h) * N, N)], as_t)
    pltpu.sync_copy(attn.at[pl.ds((4 * d + 2 + h) * N, N)], ad_t)

    zz = jnp.zeros((16,), F32)
    i16 = lax.iota(jnp.int32, 16)
    for k in range(5):
        for j in range(8):
            io_t[k, pl.ds(16 * j, 16)] = i16 + (128 * k + 16 * j)

    def zden(r, c):
        den_t[r, :] = zz
        return c

    lax.fori_loop(0, DN, zden, 0)

    @pl.when(tid == 0)
    def _():
        pltpu.sync_copy(den_t, sh_den)

    plsc.subcore_barrier()

    def aloop(i, c):
        sl = pl.ds(i * 16, 16)
        s_v = src_t[sl]
        d_v = dst_t[sl]
        a_s = plsc.load_gather(as_t, [s_v])
        a_d = plsc.load_gather(ad_t, [d_v])
        al = a_s + a_d + ae_t[sl]
        al = jnp.where(al > 0, al, NEG * al)
        ex = jnp.exp(al)
        ae_t[sl] = ex
        plsc.addupdate_scatter(den_t, [d_v >> 4, d_v & 15], ex)
        return c

    lax.fori_loop(0, ET // 16, aloop, 0)

    for k in range(5):
        pltpu.sync_copy(den_t.at[pl.ds(k * 128, 128)], sh_den.at[io_t.at[k]],
                        add=True)
    plsc.subcore_barrier()
    pltpu.sync_copy(sh_den, den_t)

    def wloop(i, c):
        sl = pl.ds(i * 16, 16)
        d_v = dst_t[sl]
        dn = plsc.load_gather(den_t, [d_v >> 4, d_v & 15])
        ae_t[sl] = ae_t[sl] / (dn + 1e-16)
        return c

    lax.fori_loop(0, ET // 16, wloop, 0)

    pltpu.sync_copy(ae_t, out.at[pl.ds(h * E + e0, ET)])


def _att_sc(d, g, attn, aeT, src, dst):
    mesh = plsc.VectorSubcoreMesh(core_axis_name="c", subcore_axis_name="s",
                                  num_cores=2, num_subcores=NS)
    fn = pl.kernel(
        functools.partial(_att_sc_body, d, g),
        out_type=jax.ShapeDtypeStruct((2 * E,), F32),
        mesh=mesh,
        scratch_types=[
            pltpu.VMEM((ET,), jnp.int32),
            pltpu.VMEM((ET,), jnp.int32),
            pltpu.VMEM((ET,), F32),
            pltpu.VMEM((N,), F32),
            pltpu.VMEM((N,), F32),
            pltpu.VMEM((DN, 16), F32),
            pltpu.VMEM((5, 128), jnp.int32),
            pltpu.VMEM_SHARED((DN, 16), F32),
        ],
        compiler_params=pltpu.CompilerParams(needs_layout_passes=False,
                                             use_tc_tiling_on_sc=False),
    )
    return fn(attn, aeT, src, dst)


def _msg_sc_body(xl0, xl1, w2, gsrcp, sdstp, out,
                 w_t, gidx, sidx, rows_t, sh_out, sem):
    h = lax.axis_index("c")
    tid = lax.axis_index("s")

    pltpu.sync_copy(gsrcp.at[tid], gidx)
    pltpu.sync_copy(sdstp.at[tid], sidx)
    pltpu.sync_copy(w2.at[pl.ds(h * E + tid * ET, ET)], w_t.at[pl.ds(0, ET)])

    zz = jnp.zeros((16,), F32)
    for j in range((ETP + 16 - ET) // 16):
        w_t[pl.ds(ET + 16 * j, 16)] = zz

    def zrow(r, c):
        for j in range(8):
            rows_t[r, pl.ds(16 * j, 16)] = zz
        return c

    lax.fori_loop(0, K, zrow, 0)
    for k in range(9):
        pltpu.sync_copy(rows_t, sh_out.at[pl.ds(tid * 625 + k * K, K)])
    pltpu.sync_copy(rows_t.at[pl.ds(0, 49)],
                    sh_out.at[pl.ds(tid * 625 + 9 * K, 49)])

    @pl.when(tid == 0)
    def _():
        pltpu.sync_copy(rows_t.at[pl.ds(0, 8)], sh_out.at[pl.ds(N, 8)])

    plsc.subcore_barrier()

    def mainloop(xl):
        def chunk(ch, c):
            pltpu.async_copy(xl.at[gidx.at[ch]], rows_t, sem).wait()

            def scale(r, c2):
                w = w_t[pl.ds(ch * K + r, 16)][0]
                for j in range(8):
                    sl = pl.ds(16 * j, 16)
                    rows_t[r, sl] = rows_t[r, sl] * w
                return c2

            lax.fori_loop(0, K, scale, 0)
            pltpu.sync_copy(rows_t, sh_out.at[sidx.at[ch]], add=True)
            return c

        lax.fori_loop(0, NCH, chunk, 0)

    @pl.when(h == 0)
    def _():
        mainloop(xl0)

    @pl.when(h == 1)
    def _():
        mainloop(xl1)

    plsc.subcore_barrier()
    pltpu.sync_copy(sh_out.at[pl.ds(tid * 624, 624)],
                    out.at[pl.ds(h * N + tid * 624, 624)])

    @pl.when(tid == NS - 1)
    def _():
        pltpu.sync_copy(sh_out.at[pl.ds(9984, 16)],
                        out.at[pl.ds(h * N + 9984, 16)])


def _msg_sc(xl0, xl1, w2, gsrcp, sdstp):
    mesh = plsc.VectorSubcoreMesh(core_axis_name="c", subcore_axis_name="s",
                                  num_cores=2, num_subcores=NS)
    fn = pl.kernel(
        _msg_sc_body,
        out_type=jax.ShapeDtypeStruct((2 * N, C), F32),
        mesh=mesh,
        scratch_types=[
            pltpu.VMEM((ETP + 16,), F32),
            pltpu.VMEM((NCH, K), jnp.int32),
            pltpu.VMEM((NCH, K), jnp.int32),
            pltpu.VMEM((K, C), F32),
            pltpu.VMEM_SHARED((N + 8, C), F32),
            pltpu.SemaphoreType.DMA,
        ],
        compiler_params=pltpu.CompilerParams(needs_layout_passes=False,
                                             use_tc_tiling_on_sc=False),
    )
    return fn(xl0, xl1, w2, gsrcp, sdstp)


def _gat_sc(d, g, xl0, xl1, attn, aeT, src, dst, gsrcp, sdstp):
    w2 = _att_sc(d, g, attn, aeT, src, dst)
    return _msg_sc(xl0, xl1, w2, gsrcp, sdstp)


def kernel(x, fwd_edges_index, bwd_edges_index, edge_attr,
           g0f_W, g0f_as, g0f_ad, g0f_eW, g0f_ae, g0f_b,
           g0b_W, g0b_as, g0b_ad, g0b_eW, g0b_ae, g0b_b,
           m0_W, m0_b, bn0_g, bn0_b, r0_W, r0_b,
           g1f_W, g1f_as, g1f_ad, g1f_eW, g1f_ae, g1f_b,
           g1b_W, g1b_as, g1b_ad, g1b_eW, g1b_ae, g1b_b,
           m1_W, m1_b, bn1_g, bn1_b, r1_W, r1_b):
    eif = fwd_edges_index.astype(jnp.int32)
    eib = bwd_edges_index.astype(jnp.int32)

    def pad_idx(v, fill):
        v2 = v.reshape(NS, ET)
        pad = jnp.full((NS, ETP - ET), fill, jnp.int32)
        return jnp.concatenate([v2, pad], axis=1).reshape(NS, NCH, K)

    srcf, dstf = eif[0], eif[1]
    srcb, dstb = eib[0], eib[1]
    gsrcf, sdstf = pad_idx(srcf, 0), pad_idx(dstf, N)
    gsrcb, sdstb = pad_idx(srcb, 0), pad_idx(dstb, N)

    aeT = _edge_attn(edge_attr, g0f_eW, g0f_ae, g0b_eW, g0b_ae,
                     g1f_eW, g1f_ae, g1b_eW, g1b_ae).reshape(-1)

    params = [
        (g0f_W, g0f_as, g0f_ad, g0f_b, g0b_W, g0b_as, g0b_ad, g0b_b,
         m0_W, m0_b, bn0_g, bn0_b, r0_W, r0_b),
        (g1f_W, g1f_as, g1f_ad, g1f_b, g1b_W, g1b_as, g1b_ad, g1b_b,
         m1_W, m1_b, bn1_g, bn1_b, r1_W, r1_b),
    ]

    xcur = x
    for l in range(2):
        (wf, asf, adf, bf, wb, asb, adb, bb,
         mw, mb, bng, bnb, rw, rb) = params[l]
        f0, f1, b0, b1 = _xl_proj(xcur, wf, wb)
        attn = _attn_proj(xcur, wf, asf, adf, wb, asb, adb).reshape(-1)
        fx = _gat_sc(0, 2 * l, f0, f1, attn, aeT, srcf, dstf, gsrcf, sdstf)
        bx = _gat_sc(1, 2 * l + 1, b0, b1, attn, aeT, srcb, dstb, gsrcb, sdstb)
        h, st = _merge_a(fx[:N], fx[N:], bx[:N], bx[N:], bf, bb, mw, mb)
        if xcur.shape[1] != HC:
            xcur = _merge_b(h, st, bng, bnb, xcur, rw, rb)
        else:
            xcur = _merge_b(h, st, bng, bnb, xcur)
    return xcur
